# baseline (speedup 1.0000x reference)
import functools

import jax
import jax.numpy as jnp
from jax import lax
from jax.experimental import pallas as pl
from jax.experimental.pallas import tpu as pltpu
from jax.experimental.pallas import tpu_sc as plsc

_G = 64
_NC = 2
_NS = 16
_NW = _NC * _NS
_CH = 64
_NB = 4
_PH = 40
_PADROWS = 48


def _chunks(total, step):
    out, off = [], 0
    while off < total:
        out.append((off, min(step, total - off)))
        off += min(step, total - off)
    return out


def _sc_mesh():
    return plsc.VectorSubcoreMesh(core_axis_name="c", subcore_axis_name="s")


def _sc_degree(dst3, zflat, ones_v, n_acc, dpt, k_chunks, dc):
    ph0 = -(-(k_chunks // 2) // 8) * 8
    phases = [(0, ph0), (ph0, k_chunks - ph0)]
    _DW = 8

    @functools.partial(
        pl.kernel,
        out_type=jax.ShapeDtypeStruct((_NC * n_acc,), jnp.float32),
        mesh=_sc_mesh(),
        scratch_types=[
            pltpu.VMEM((ph0, dc), jnp.int32),
            pltpu.VMEM((dpt,), jnp.float32),
            pltpu.VMEM((dc,), jnp.float32),
            pltpu.VMEM_SHARED((n_acc,), jnp.float32),
            pltpu.SemaphoreType.DMA,
        ],
    )
    def deg_kernel(dst_hbm, z_hbm, ones_hbm, deg_hbm, dbuf, fbuf, ones_b, acc,
                   sem):
        c = lax.axis_index("c")
        s = lax.axis_index("s")
        wid = s * _NC + c
        pltpu.sync_copy(z_hbm, fbuf)
        pltpu.sync_copy(fbuf, acc.at[pl.ds(s * dpt, dpt)])
        pltpu.sync_copy(ones_hbm, ones_b)
        plsc.subcore_barrier()

        def step(j, carry):
            pltpu.async_copy(ones_b, acc.at[dbuf.at[j]], sem, add=True)

            @pl.when(j >= _DW)
            def _():
                pltpu.make_async_copy(ones_b, acc.at[dbuf.at[j]], sem).wait()
            return carry

        for p0, pn in phases:
            pltpu.sync_copy(dst_hbm.at[wid, pl.ds(p0, pn)],
                            dbuf.at[pl.ds(0, pn)])
            lax.fori_loop(0, pn, step, 0)
            for _ in range(min(_DW, pn)):
                pltpu.make_async_copy(ones_b, acc.at[dbuf.at[0]], sem).wait()
        plsc.subcore_barrier()
        pltpu.sync_copy(acc.at[pl.ds(s * dpt, dpt)], fbuf)
        pltpu.sync_copy(fbuf, deg_hbm.at[pl.ds(c * n_acc + s * dpt, dpt)])

    return deg_kernel(dst3, zflat, ones_v)


def _sc_spmm(p, src3, dst3, n, h, n_acc, dpt, k_chunks):
    drt = -(-(-(-n // _NS)) // 8) * 8
    assert drt * _NS <= n_acc
    dump_chunks = _chunks(drt, _CH)
    zero_chunks = _chunks(dpt, _CH)

    assert k_chunks % _PH == 0 and _PH % _NB == 0
    n_phases = k_chunks // _PH

    @functools.partial(
        pl.kernel,
        out_type=jax.ShapeDtypeStruct((_NC, drt * _NS, h), jnp.float32),
        mesh=_sc_mesh(),
        scratch_types=[
            pltpu.VMEM((_PH, _CH), jnp.int32),
            pltpu.VMEM((_PH, _CH), jnp.int32),
            [pltpu.VMEM((_CH, h), jnp.float32) for _ in range(_NB)],
            pltpu.VMEM_SHARED((n_acc, h), jnp.float32),
            [pltpu.SemaphoreType.DMA for _ in range(_NB)],
            [pltpu.SemaphoreType.DMA for _ in range(_NB)],
        ],
    )
    def spmm_kernel(p_hbm, src_hbm, dst_hbm, s_hbm,
                    sbuf, dbuf, rows, acc, gsem, ssem):
        c = lax.axis_index("c")
        s = lax.axis_index("s")
        wid = s * _NC + c

        def zstep(r, carry):
            for cc in range(h // 16):
                rows[0][r, pl.ds(cc * 16, 16)] = jnp.zeros((16,), jnp.float32)
            return carry

        lax.fori_loop(0, _CH, zstep, 0)
        zdesc = [pltpu.async_copy(rows[0].at[pl.ds(0, zl)],
                                  acc.at[pl.ds(s * dpt + zo, zl)], gsem[0])
                 for zo, zl in zero_chunks]
        pltpu.sync_copy(src_hbm.at[wid, pl.ds(0, _PH)], sbuf)
        pltpu.sync_copy(dst_hbm.at[wid, pl.ds(0, _PH)], dbuf)
        for d in zdesc:
            d.wait()
        for k in range(2):
            pltpu.async_copy(p_hbm.at[sbuf.at[k]], rows[k], gsem[k])
        plsc.subcore_barrier()

        def body(m, carry):
            for k in range(_NB):
                j = _NB * m + k
                pltpu.make_async_copy(p_hbm.at[sbuf.at[j]], rows[k],
                                      gsem[k]).wait()
                pltpu.async_copy(rows[k], acc.at[dbuf.at[j]], ssem[k],
                                 add=True)
                kn = (k + 2) % _NB

                @pl.when(jnp.logical_and(j >= 2, j + 2 < _PH))
                def _():
                    pltpu.make_async_copy(rows[kn], acc.at[dbuf.at[j]],
                                          ssem[kn]).wait()
                    pltpu.async_copy(p_hbm.at[sbuf.at[j + 2]], rows[kn],
                                     gsem[kn])

                if k < 2:
                    @pl.when(j < 2)
                    def _():
                        pltpu.async_copy(p_hbm.at[sbuf.at[j + 2]], rows[kn],
                                         gsem[kn])
            return carry

        for p0 in range(0, k_chunks, _PH):
            if p0 > 0:
                pltpu.sync_copy(src_hbm.at[wid, pl.ds(p0, _PH)], sbuf)
                pltpu.sync_copy(dst_hbm.at[wid, pl.ds(p0, _PH)], dbuf)
                for k in range(2):
                    pltpu.async_copy(p_hbm.at[sbuf.at[k]], rows[k], gsem[k])
            lax.fori_loop(0, _PH // _NB, body, 0)
            for k in range(_NB):
                pltpu.make_async_copy(rows[k], acc.at[dbuf.at[0]],
                                      ssem[k]).wait()
        plsc.subcore_barrier()
        base = s * drt
        pend = [None, None]
        for i, (off2, ln) in enumerate(dump_chunks):
            b = i % 2
            if pend[b] is not None:
                pend[b].wait()
            pltpu.async_copy(acc.at[pl.ds(base + off2, ln)],
                             rows[b].at[pl.ds(0, ln)], gsem[b]).wait()
            pend[b] = pltpu.async_copy(rows[b].at[pl.ds(0, ln)],
                                       s_hbm.at[c, pl.ds(base + off2, ln)],
                                       ssem[b])
        for d in pend:
            if d is not None:
                d.wait()

    return spmm_kernel(p, src3, dst3)


def kernel(x, edge_index, batch, W1, b1, W2, b2, Wc1, bc1, Wc2, bc2):
    N, D = x.shape
    H = W1.shape[1]
    OUT = Wc2.shape[1]
    E = edge_index.shape[1]
    f32 = jnp.float32

    epw = -(-E // (_NW * _PH * _CH)) * _PH * _CH
    k_chunks = epw // _CH
    epad = epw * _NW
    dpt = -(-(N + _PADROWS) // (_NS * 8)) * 8
    n_acc = dpt * _NS

    pad = epad - E
    ar = jnp.arange(pad, dtype=jnp.int32)
    src_flat = jnp.concatenate([edge_index[0], ar % N])
    dst_flat = jnp.concatenate([edge_index[1], N + ar % (n_acc - N)])
    src_p = src_flat.reshape(_NW, k_chunks, _CH)
    dst_p = dst_flat.reshape(_NW, k_chunks, _CH)
    zflat = jnp.zeros((dpt,), f32)
    dc = 128
    dst_d = dst_flat.reshape(_NW, epw // dc, dc)
    ones_v = jnp.ones((dc,), f32)

    degp = _sc_degree(dst_d, zflat, ones_v, n_acc, dpt, epw // dc, dc)
    deg3 = degp.reshape(_NC, n_acc, 1)

    BR = 2000
    grid = (N // BR,)

    def t1a_body(x_ref, w_ref, o_ref):
        o_ref[...] = jnp.dot(x_ref[...], w_ref[...],
                             preferred_element_type=f32)

    h1 = pl.pallas_call(
        t1a_body,
        grid=grid,
        in_specs=[
            pl.BlockSpec((BR, D), lambda i: (i, 0)),
            pl.BlockSpec((D, H), lambda i: (0, 0)),
        ],
        out_specs=pl.BlockSpec((BR, H), lambda i: (i, 0)),
        out_shape=jax.ShapeDtypeStruct((N, H), f32),
    )(x, W1)

    def t1_body(deg_ref, h_ref, o_ref):
        dinv = lax.rsqrt(deg_ref[0] + deg_ref[1] + 1.0)
        o_ref[...] = h_ref[...] * dinv

    p1 = pl.pallas_call(
        t1_body,
        grid=grid,
        in_specs=[
            pl.BlockSpec((_NC, BR, 1), lambda i: (0, i, 0)),
            pl.BlockSpec((BR, H), lambda i: (i, 0)),
        ],
        out_specs=pl.BlockSpec((BR, H), lambda i: (i, 0)),
        out_shape=jax.ShapeDtypeStruct((N, H), f32),
    )(deg3, h1)

    s1 = _sc_spmm(p1, src_p, dst_p, N, H, n_acc, dpt, k_chunks)

    def t2_body(deg_ref, s_ref, p_ref, w_ref, b_ref, o_ref):
        dinv = lax.rsqrt(deg_ref[0] + deg_ref[1] + 1.0)
        t = dinv * (s_ref[0] + s_ref[1] + p_ref[...]) + b_ref[...]
        a = jnp.maximum(t, 0.0)
        o_ref[...] = jnp.dot(a, w_ref[...], preferred_element_type=f32) * dinv

    p2 = pl.pallas_call(
        t2_body,
        grid=grid,
        in_specs=[
            pl.BlockSpec((_NC, BR, 1), lambda i: (0, i, 0)),
            pl.BlockSpec((_NC, BR, H), lambda i: (0, i, 0)),
            pl.BlockSpec((BR, H), lambda i: (i, 0)),
            pl.BlockSpec((H, H), lambda i: (0, 0)),
            pl.BlockSpec((1, H), lambda i: (0, 0)),
        ],
        out_specs=pl.BlockSpec((BR, H), lambda i: (i, 0)),
        out_shape=jax.ShapeDtypeStruct((N, H), f32),
    )(deg3, s1, p1, W2, b1.reshape(1, H))

    s2 = _sc_spmm(p2, src_p, dst_p, N, H, n_acc, dpt, k_chunks)

    nblk = grid[0]

    def t3_body(deg_ref, s_ref, p_ref, b_ref, bat_ref,
                wc1_ref, bc1_ref, wc2_ref, bc2_ref,
                h_ref, gr_ref, lg_ref, sums, cnts):
        i = pl.program_id(0)
        dinv = lax.rsqrt(deg_ref[0] + deg_ref[1] + 1.0)
        hblk = dinv * (s_ref[0] + s_ref[1] + p_ref[...]) + b_ref[...]
        h_ref[...] = hblk
        gids = lax.broadcasted_iota(jnp.int32, (1, _G), 1)
        onehot = (bat_ref[...] == gids).astype(f32)
        dn = (((0,), (0,)), ((), ()))
        ps = lax.dot_general(onehot, hblk, dn, preferred_element_type=f32)
        pc = lax.dot_general(onehot, jnp.ones((BR, H), f32), dn,
                             preferred_element_type=f32)

        @pl.when(i == 0)
        def _():
            sums[...] = ps
            cnts[...] = pc

        @pl.when(i > 0)
        def _():
            sums[...] += ps
            cnts[...] += pc

        @pl.when(i == nblk - 1)
        def _():
            gr = sums[...] / jnp.maximum(cnts[...], 1.0)
            gr_ref[...] = gr
            z = jnp.maximum(
                jnp.dot(gr, wc1_ref[...], preferred_element_type=f32)
                + bc1_ref[...], 0.0)
            lg_ref[...] = (jnp.dot(z, wc2_ref[...], preferred_element_type=f32)
                           + bc2_ref[...])

    h, graph_reps, logits = pl.pallas_call(
        t3_body,
        grid=grid,
        in_specs=[
            pl.BlockSpec((_NC, BR, 1), lambda i: (0, i, 0)),
            pl.BlockSpec((_NC, BR, H), lambda i: (0, i, 0)),
            pl.BlockSpec((BR, H), lambda i: (i, 0)),
            pl.BlockSpec((1, H), lambda i: (0, 0)),
            pl.BlockSpec((BR, 1), lambda i: (i, 0)),
            pl.BlockSpec((H, H), lambda i: (0, 0)),
            pl.BlockSpec((1, H), lambda i: (0, 0)),
            pl.BlockSpec((H, OUT), lambda i: (0, 0)),
            pl.BlockSpec((1, OUT), lambda i: (0, 0)),
        ],
        out_specs=[
            pl.BlockSpec((BR, H), lambda i: (i, 0)),
            pl.BlockSpec((_G, H), lambda i: (0, 0)),
            pl.BlockSpec((_G, OUT), lambda i: (0, 0)),
        ],
        out_shape=[
            jax.ShapeDtypeStruct((N, H), f32),
            jax.ShapeDtypeStruct((_G, H), f32),
            jax.ShapeDtypeStruct((_G, OUT), f32),
        ],
        scratch_shapes=[
            pltpu.VMEM((_G, H), f32),
            pltpu.VMEM((_G, H), f32),
        ],
    )(deg3, s2, p2, b2.reshape(1, H), batch.reshape(N, 1),
      Wc1, bc1.reshape(1, H), Wc2, bc2.reshape(1, OUT))

    return (h, graph_reps, logits)


# speedup vs baseline: 1.0025x; 1.0025x over previous
import functools

import jax
import jax.numpy as jnp
from jax import lax
from jax.experimental import pallas as pl
from jax.experimental.pallas import tpu as pltpu
from jax.experimental.pallas import tpu_sc as plsc

_G = 64
_NC = 2
_NS = 16
_NW = _NC * _NS
_CH = 64
_NB = 4
_PH = 40
_PADROWS = 48


def _chunks(total, step):
    out, off = [], 0
    while off < total:
        out.append((off, min(step, total - off)))
        off += min(step, total - off)
    return out


def _sc_mesh():
    return plsc.VectorSubcoreMesh(core_axis_name="c", subcore_axis_name="s")


def _sc_degree(dst3, zflat, ones_v, n_acc, dpt, k_chunks, dc):
    ph0 = -(-(k_chunks // 2) // 8) * 8
    phases = [(0, ph0), (ph0, k_chunks - ph0)]
    _DW = 8

    @functools.partial(
        pl.kernel,
        out_type=jax.ShapeDtypeStruct((_NC * n_acc,), jnp.float32),
        mesh=_sc_mesh(),
        scratch_types=[
            pltpu.VMEM((ph0, dc), jnp.int32),
            pltpu.VMEM((dpt,), jnp.float32),
            pltpu.VMEM((dc,), jnp.float32),
            pltpu.VMEM_SHARED((n_acc,), jnp.float32),
            pltpu.SemaphoreType.DMA,
        ],
    )
    def deg_kernel(dst_hbm, z_hbm, ones_hbm, deg_hbm, dbuf, fbuf, ones_b, acc,
                   sem):
        c = lax.axis_index("c")
        s = lax.axis_index("s")
        wid = s * _NC + c
        pltpu.sync_copy(z_hbm, fbuf)
        pltpu.sync_copy(fbuf, acc.at[pl.ds(s * dpt, dpt)])
        pltpu.sync_copy(ones_hbm, ones_b)
        plsc.subcore_barrier()

        def step(j, carry):
            pltpu.async_copy(ones_b, acc.at[dbuf.at[j]], sem, add=True)

            @pl.when(j >= _DW)
            def _():
                pltpu.make_async_copy(ones_b, acc.at[dbuf.at[j]], sem).wait()
            return carry

        for p0, pn in phases:
            pltpu.sync_copy(dst_hbm.at[wid, pl.ds(p0, pn)],
                            dbuf.at[pl.ds(0, pn)])
            lax.fori_loop(0, pn, step, 0)
            for _ in range(min(_DW, pn)):
                pltpu.make_async_copy(ones_b, acc.at[dbuf.at[0]], sem).wait()
        plsc.subcore_barrier()
        pltpu.sync_copy(acc.at[pl.ds(s * dpt, dpt)], fbuf)
        pltpu.sync_copy(fbuf, deg_hbm.at[pl.ds(c * n_acc + s * dpt, dpt)])

    return deg_kernel(dst3, zflat, ones_v)


def _sc_spmm(p, src3, dst3, n, h, n_acc, dpt, k_chunks):
    drt = -(-(-(-n // _NS)) // 8) * 8
    assert drt * _NS <= n_acc
    dump_chunks = _chunks(drt, _CH)
    zero_chunks = _chunks(dpt, _CH)

    assert k_chunks % _PH == 0 and _PH % _NB == 0
    n_phases = k_chunks // _PH

    @functools.partial(
        pl.kernel,
        out_type=jax.ShapeDtypeStruct((_NC, drt * _NS, h), jnp.float32),
        mesh=_sc_mesh(),
        scratch_types=[
            pltpu.VMEM((_PH, _CH), jnp.int32),
            pltpu.VMEM((_PH, _CH), jnp.int32),
            [pltpu.VMEM((_CH, h), jnp.float32) for _ in range(_NB)],
            pltpu.VMEM_SHARED((n_acc, h), jnp.float32),
            [pltpu.SemaphoreType.DMA for _ in range(_NB)],
            [pltpu.SemaphoreType.DMA for _ in range(_NB)],
        ],
    )
    def spmm_kernel(p_hbm, src_hbm, dst_hbm, s_hbm,
                    sbuf, dbuf, rows, acc, gsem, ssem):
        c = lax.axis_index("c")
        s = lax.axis_index("s")
        wid = s * _NC + c

        def zstep(r, carry):
            for cc in range(h // 16):
                rows[0][r, pl.ds(cc * 16, 16)] = jnp.zeros((16,), jnp.float32)
            return carry

        lax.fori_loop(0, _CH, zstep, 0)
        zdesc = [pltpu.async_copy(rows[0].at[pl.ds(0, zl)],
                                  acc.at[pl.ds(s * dpt + zo, zl)], gsem[0])
                 for zo, zl in zero_chunks]
        pltpu.sync_copy(src_hbm.at[wid, pl.ds(0, _PH)], sbuf)
        pltpu.sync_copy(dst_hbm.at[wid, pl.ds(0, _PH)], dbuf)
        for d in zdesc:
            d.wait()
        for k in range(2):
            pltpu.async_copy(p_hbm.at[sbuf.at[k]], rows[k], gsem[k])
        plsc.subcore_barrier()

        def stage(j, k, do_wait, do_gather):
            pltpu.make_async_copy(p_hbm.at[sbuf.at[j]], rows[k],
                                  gsem[k]).wait()
            pltpu.async_copy(rows[k], acc.at[dbuf.at[j]], ssem[k], add=True)
            kn = (k + 2) % _NB
            if do_wait:
                pltpu.make_async_copy(rows[kn], acc.at[dbuf.at[j]],
                                      ssem[kn]).wait()
            if do_gather:
                pltpu.async_copy(p_hbm.at[sbuf.at[j + 2]], rows[kn], gsem[kn])

        def body(m, carry):
            for k in range(_NB):
                stage(_NB * m + k, k, True, True)
            return carry

        M = _PH // _NB
        for p0 in range(0, k_chunks, _PH):
            if p0 > 0:
                pltpu.sync_copy(src_hbm.at[wid, pl.ds(p0, _PH)], sbuf)
                pltpu.sync_copy(dst_hbm.at[wid, pl.ds(p0, _PH)], dbuf)
                for k in range(2):
                    pltpu.async_copy(p_hbm.at[sbuf.at[k]], rows[k], gsem[k])
            for k in range(_NB):
                stage(k, k, k >= 2, True)
            lax.fori_loop(1, M - 1, body, 0)
            for k in range(_NB):
                stage(_PH - _NB + k, k, k < 2, k < 2)
            for k in range(_NB):
                pltpu.make_async_copy(rows[k], acc.at[dbuf.at[0]],
                                      ssem[k]).wait()
        plsc.subcore_barrier()
        base = s * drt
        pend = [None, None]
        for i, (off2, ln) in enumerate(dump_chunks):
            b = i % 2
            if pend[b] is not None:
                pend[b].wait()
            pltpu.async_copy(acc.at[pl.ds(base + off2, ln)],
                             rows[b].at[pl.ds(0, ln)], gsem[b]).wait()
            pend[b] = pltpu.async_copy(rows[b].at[pl.ds(0, ln)],
                                       s_hbm.at[c, pl.ds(base + off2, ln)],
                                       ssem[b])
        for d in pend:
            if d is not None:
                d.wait()

    return spmm_kernel(p, src3, dst3)


def kernel(x, edge_index, batch, W1, b1, W2, b2, Wc1, bc1, Wc2, bc2):
    N, D = x.shape
    H = W1.shape[1]
    OUT = Wc2.shape[1]
    E = edge_index.shape[1]
    f32 = jnp.float32

    epw = -(-E // (_NW * _PH * _CH)) * _PH * _CH
    k_chunks = epw // _CH
    epad = epw * _NW
    dpt = -(-(N + _PADROWS) // (_NS * 8)) * 8
    n_acc = dpt * _NS

    pad = epad - E
    ar = jnp.arange(pad, dtype=jnp.int32)
    src_flat = jnp.concatenate([edge_index[0], ar % N])
    dst_flat = jnp.concatenate([edge_index[1], N + ar % (n_acc - N)])
    src_p = src_flat.reshape(_NW, k_chunks, _CH)
    dst_p = dst_flat.reshape(_NW, k_chunks, _CH)
    zflat = jnp.zeros((dpt,), f32)
    dc = 128
    dst_d = dst_flat.reshape(_NW, epw // dc, dc)
    ones_v = jnp.ones((dc,), f32)

    degp = _sc_degree(dst_d, zflat, ones_v, n_acc, dpt, epw // dc, dc)
    deg3 = degp.reshape(_NC, n_acc, 1)

    BR = 2000
    grid = (N // BR,)

    def t1_body(deg_ref, x_ref, w_ref, o_ref):
        dinv = lax.rsqrt(deg_ref[0] + deg_ref[1] + 1.0)
        o_ref[...] = jnp.dot(x_ref[...], w_ref[...],
                             preferred_element_type=f32) * dinv

    p1 = pl.pallas_call(
        t1_body,
        grid=grid,
        in_specs=[
            pl.BlockSpec((_NC, BR, 1), lambda i: (0, i, 0)),
            pl.BlockSpec((BR, D), lambda i: (i, 0)),
            pl.BlockSpec((D, H), lambda i: (0, 0)),
        ],
        out_specs=pl.BlockSpec((BR, H), lambda i: (i, 0)),
        out_shape=jax.ShapeDtypeStruct((N, H), f32),
    )(deg3, x, W1)

    s1 = _sc_spmm(p1, src_p, dst_p, N, H, n_acc, dpt, k_chunks)

    def t2_body(deg_ref, s_ref, p_ref, w_ref, b_ref, o_ref):
        dinv = lax.rsqrt(deg_ref[0] + deg_ref[1] + 1.0)
        t = dinv * (s_ref[0] + s_ref[1] + p_ref[...]) + b_ref[...]
        a = jnp.maximum(t, 0.0)
        o_ref[...] = jnp.dot(a, w_ref[...], preferred_element_type=f32) * dinv

    p2 = pl.pallas_call(
        t2_body,
        grid=grid,
        in_specs=[
            pl.BlockSpec((_NC, BR, 1), lambda i: (0, i, 0)),
            pl.BlockSpec((_NC, BR, H), lambda i: (0, i, 0)),
            pl.BlockSpec((BR, H), lambda i: (i, 0)),
            pl.BlockSpec((H, H), lambda i: (0, 0)),
            pl.BlockSpec((1, H), lambda i: (0, 0)),
        ],
        out_specs=pl.BlockSpec((BR, H), lambda i: (i, 0)),
        out_shape=jax.ShapeDtypeStruct((N, H), f32),
    )(deg3, s1, p1, W2, b1.reshape(1, H))

    s2 = _sc_spmm(p2, src_p, dst_p, N, H, n_acc, dpt, k_chunks)

    nblk = grid[0]

    def t3_body(deg_ref, s_ref, p_ref, b_ref, bat_ref,
                wc1_ref, bc1_ref, wc2_ref, bc2_ref,
                h_ref, gr_ref, lg_ref, sums, cnts):
        i = pl.program_id(0)
        dinv = lax.rsqrt(deg_ref[0] + deg_ref[1] + 1.0)
        hblk = dinv * (s_ref[0] + s_ref[1] + p_ref[...]) + b_ref[...]
        h_ref[...] = hblk
        gids = lax.broadcasted_iota(jnp.int32, (1, _G), 1)
        onehot = (bat_ref[...] == gids).astype(f32)
        dn = (((0,), (0,)), ((), ()))
        ps = lax.dot_general(onehot, hblk, dn, preferred_element_type=f32)
        pc = lax.dot_general(onehot, jnp.ones((BR, H), f32), dn,
                             preferred_element_type=f32)

        @pl.when(i == 0)
        def _():
            sums[...] = ps
            cnts[...] = pc

        @pl.when(i > 0)
        def _():
            sums[...] += ps
            cnts[...] += pc

        @pl.when(i == nblk - 1)
        def _():
            gr = sums[...] / jnp.maximum(cnts[...], 1.0)
            gr_ref[...] = gr
            z = jnp.maximum(
                jnp.dot(gr, wc1_ref[...], preferred_element_type=f32)
                + bc1_ref[...], 0.0)
            lg_ref[...] = (jnp.dot(z, wc2_ref[...], preferred_element_type=f32)
                           + bc2_ref[...])

    h, graph_reps, logits = pl.pallas_call(
        t3_body,
        grid=grid,
        in_specs=[
            pl.BlockSpec((_NC, BR, 1), lambda i: (0, i, 0)),
            pl.BlockSpec((_NC, BR, H), lambda i: (0, i, 0)),
            pl.BlockSpec((BR, H), lambda i: (i, 0)),
            pl.BlockSpec((1, H), lambda i: (0, 0)),
            pl.BlockSpec((BR, 1), lambda i: (i, 0)),
            pl.BlockSpec((H, H), lambda i: (0, 0)),
            pl.BlockSpec((1, H), lambda i: (0, 0)),
            pl.BlockSpec((H, OUT), lambda i: (0, 0)),
            pl.BlockSpec((1, OUT), lambda i: (0, 0)),
        ],
        out_specs=[
            pl.BlockSpec((BR, H), lambda i: (i, 0)),
            pl.BlockSpec((_G, H), lambda i: (0, 0)),
            pl.BlockSpec((_G, OUT), lambda i: (0, 0)),
        ],
        out_shape=[
            jax.ShapeDtypeStruct((N, H), f32),
            jax.ShapeDtypeStruct((_G, H), f32),
            jax.ShapeDtypeStruct((_G, OUT), f32),
        ],
        scratch_shapes=[
            pltpu.VMEM((_G, H), f32),
            pltpu.VMEM((_G, H), f32),
        ],
    )(deg3, s2, p2, b2.reshape(1, H), batch.reshape(N, 1),
      Wc1, bc1.reshape(1, H), Wc2, bc2.reshape(1, OUT))

    return (h, graph_reps, logits)
